# baseline (speedup 1.0000x reference)
import jax
import jax.numpy as jnp
from jax.experimental import pallas as pl
from jax.experimental.pallas import tpu as pltpu

_ALPHA = 0.1
_BETA = 0.5
_NB = 1000


def _fused(x_ref, incl_ref, incr_ref, wT_ref, out_ref, macc_ref, msc_ref):
    i = pl.program_id(0)
    g = pl.num_programs(0) // 2
    nb = incl_ref.shape[0]
    eh = incl_ref.shape[1]
    d = x_ref.shape[1]
    row0 = jnp.where(i < g, i, i - g) * nb
    xb = x_ref[pl.ds(row0, nb), :]
    bl = incl_ref[...]
    br = incr_ref[...]
    nd = (jnp.sum(bl, axis=1, keepdims=True)
          + jnp.sum(br, axis=1, keepdims=True))
    bbl = bl.astype(jnp.bfloat16)
    bbr = br.astype(jnp.bfloat16)

    @pl.when(i < g)
    def _pass1():
        lhs = jnp.concatenate(
            [jnp.transpose(xb).astype(jnp.bfloat16),
             jnp.ones((1, nb), jnp.bfloat16),
             jnp.transpose(nd).astype(jnp.bfloat16)], axis=0)
        ml = jax.lax.dot_general(lhs, bbl, (((1,), (0,)), ((), ())),
                                 preferred_element_type=jnp.float32)
        mr = jax.lax.dot_general(lhs, bbr, (((1,), (0,)), ((), ())),
                                 preferred_element_type=jnp.float32)

        @pl.when(i == 0)
        def _init():
            macc_ref[:, :eh] = ml
            macc_ref[:, eh:] = mr

        @pl.when(i != 0)
        def _acc():
            macc_ref[:, :eh] += ml
            macc_ref[:, eh:] += mr

    @pl.when(i >= g)
    def _pass2():
        @pl.when(i == g)
        def _prep():
            cs = macc_ref[d:d + 1, :]
            ws = macc_ref[d + 1:d + 2, :]
            inv_e = jax.lax.rsqrt(ws / cs)
            msc_ref[...] = (macc_ref[:d, :] * inv_e).astype(jnp.bfloat16)

        agg = (jax.lax.dot_general(bbl, msc_ref[:, :eh],
                                   (((1,), (1,)), ((), ())),
                                   preferred_element_type=jnp.float32)
               + jax.lax.dot_general(bbr, msc_ref[:, eh:],
                                     (((1,), (1,)), ((), ())),
                                     preferred_element_type=jnp.float32))
        xc = (1.0 - _ALPHA) * (agg * jax.lax.rsqrt(nd)) + _ALPHA * xb
        out_ref[...] = (1.0 - _BETA) * xc + _BETA * jax.lax.dot_general(
            xc.astype(jnp.bfloat16), wT_ref[...].astype(jnp.bfloat16),
            (((1,), (0,)), ((), ())),
            preferred_element_type=jnp.float32)


def kernel(x_0, incidence_1, W):
    N, D = x_0.shape
    E = incidence_1.shape[1]
    G = N // _NB

    def _left(i):
        return (jax.lax.select(i < G, i, i - G), 0)

    def _right(i):
        return (jax.lax.select(i < G, i, i - G), 1)

    return pl.pallas_call(
        _fused,
        grid=(2 * G,),
        in_specs=[
            pl.BlockSpec((N, D), lambda i: (0, 0)),
            pl.BlockSpec((_NB, E // 2), _left),
            pl.BlockSpec((_NB, E // 2), _right),
            pl.BlockSpec((D, D), lambda i: (0, 0)),
        ],
        out_specs=pl.BlockSpec((_NB, D), _left),
        out_shape=jax.ShapeDtypeStruct((N, D), jnp.float32),
        scratch_shapes=[
            pltpu.VMEM((D + 2, E), jnp.float32),
            pltpu.VMEM((D, E), jnp.bfloat16),
        ],
    )(x_0, incidence_1, incidence_1, W.T)


# speedup vs baseline: 1.0164x; 1.0164x over previous
import jax
import jax.numpy as jnp
from jax.experimental import pallas as pl
from jax.experimental.pallas import tpu as pltpu

_ALPHA = 0.1
_BETA = 0.5
_NB = 1000


def _fused(x_ref, inc_ref, wT_ref, out_ref, macc_ref, msc_ref):
    i = pl.program_id(0)
    g = pl.num_programs(0) // 2
    nb = inc_ref.shape[0]
    d = x_ref.shape[1]
    row0 = jnp.where(i < g, i, i - g) * nb
    xb = x_ref[pl.ds(row0, nb), :]
    blk = inc_ref[...]
    nd = jnp.sum(blk, axis=1, keepdims=True)
    bblk = blk.astype(jnp.bfloat16)

    @pl.when(i < g)
    def _pass1():
        lhs = jnp.concatenate(
            [jnp.transpose(xb).astype(jnp.bfloat16),
             jnp.ones((1, nb), jnp.bfloat16),
             jnp.transpose(nd).astype(jnp.bfloat16)], axis=0)
        m = jax.lax.dot_general(lhs, bblk, (((1,), (0,)), ((), ())),
                                preferred_element_type=jnp.float32)

        @pl.when(i == 0)
        def _init():
            macc_ref[...] = m

        @pl.when(i != 0)
        def _acc():
            macc_ref[...] += m

    @pl.when(i >= g)
    def _pass2():
        @pl.when(i == g)
        def _prep():
            cs = macc_ref[d:d + 1, :]
            ws = macc_ref[d + 1:d + 2, :]
            inv_e = jax.lax.rsqrt(ws / cs)
            msc_ref[...] = (macc_ref[:d, :] * inv_e).astype(jnp.bfloat16)

        agg = jax.lax.dot_general(bblk, msc_ref[...],
                                  (((1,), (1,)), ((), ())),
                                  preferred_element_type=jnp.float32)
        xc = (1.0 - _ALPHA) * (agg * jax.lax.rsqrt(nd)) + _ALPHA * xb
        out_ref[...] = (1.0 - _BETA) * xc + _BETA * jax.lax.dot_general(
            xc.astype(jnp.bfloat16), wT_ref[...].astype(jnp.bfloat16),
            (((1,), (0,)), ((), ())),
            preferred_element_type=jnp.float32)


def kernel(x_0, incidence_1, W):
    N, D = x_0.shape
    E = incidence_1.shape[1]
    G = N // _NB

    def _row(i):
        return (jax.lax.select(i < G, i, i - G), 0)

    return pl.pallas_call(
        _fused,
        grid=(2 * G,),
        in_specs=[
            pl.BlockSpec((N, D), lambda i: (0, 0)),
            pl.BlockSpec((_NB, E), _row),
            pl.BlockSpec((D, D), lambda i: (0, 0)),
        ],
        out_specs=pl.BlockSpec((_NB, D), _row),
        out_shape=jax.ShapeDtypeStruct((N, D), jnp.float32),
        scratch_shapes=[
            pltpu.VMEM((D + 2, E), jnp.float32),
            pltpu.VMEM((D, E), jnp.bfloat16),
        ],
    )(x_0, incidence_1, W.T)
